# Initial kernel scaffold; baseline (speedup 1.0000x reference)
import jax
import jax.numpy as jnp
from jax.experimental import pallas as pl

BB = 8


def _body(x_ref, wz_ref, lzw_ref, bz_ref, lzb_ref, wh_ref, lhw_ref, bh_ref,
          lhb_ref, att_ref, o_ref):
    p = pl.program_id(1)
    n = x_ref.shape[2]
    hidden = lzw_ref.shape[1]

    lzw1 = lzw_ref[0:hidden, :]
    lhw1 = lhw_ref[0:hidden, :]
    gmz = jnp.dot(wz_ref[...], lzw1, preferred_element_type=jnp.float32)
    gmh = jnp.dot(wh_ref[...], lhw1, preferred_element_type=jnp.float32)
    gmat = jnp.concatenate([gmz, gmh], axis=1)
    gz = jnp.dot(bz_ref[...], lzw1) + lzb_ref[...]
    gh = jnp.dot(bh_ref[...], lhw1) + lhb_ref[...]

    r = jax.lax.broadcasted_iota(jnp.float32, (n, n), 0)
    c = jax.lax.broadcasted_iota(jnp.float32, (n, n), 1)
    adj = jnp.where(r >= c,
                    jax.lax.rsqrt(r + 1.0) * jax.lax.rsqrt(c + 1.0), 0.0)

    a = att_ref[...]
    e = jnp.exp(a - jnp.max(a))
    onehot = jax.lax.broadcasted_iota(jnp.int32, a.shape, 1) == p
    prob_p = jnp.sum(jnp.where(onehot, e, 0.0)) / jnp.sum(e)

    @pl.when(p == 0)
    def _init():
        o_ref[...] = jnp.zeros_like(o_ref)

    rows = []
    for b in range(BB):
        xb = x_ref[b, 0]
        ub = jnp.dot(xb, gmat, preferred_element_type=jnp.float32)
        pb = jnp.dot(adj, ub, preferred_element_type=jnp.float32)
        pz = pb[:, 0:hidden] + gz
        ph = pb[:, hidden:2 * hidden] + gh
        hp = (1.0 - jax.nn.sigmoid(pz)) * jnp.tanh(ph)
        rows.append(jnp.mean(hp, axis=0, keepdims=True))
    o_ref[...] += prob_p * jnp.concatenate(rows, axis=0)


def kernel(agent_obs, hideout_obs, timestep_obs, num_agents, edge_index,
           Wz, bz, Wr, br, Wh, bh, LzW, Lzb, LrW, Lrb, LhW, Lhb, attention):
    agent_obs = agent_obs.astype(jnp.float32)
    batch, periods, n, f = agent_obs.shape
    hidden = LzW.shape[1]

    x = pl.pallas_call(
        _body,
        grid=(batch // BB, periods),
        in_specs=[
            pl.BlockSpec((BB, 1, n, f), lambda i, p: (i, p, 0, 0)),
            pl.BlockSpec((f, hidden), lambda i, p: (0, 0)),
            pl.BlockSpec((2 * hidden, hidden), lambda i, p: (0, 0)),
            pl.BlockSpec((1, hidden), lambda i, p: (0, 0)),
            pl.BlockSpec((1, hidden), lambda i, p: (0, 0)),
            pl.BlockSpec((f, hidden), lambda i, p: (0, 0)),
            pl.BlockSpec((2 * hidden, hidden), lambda i, p: (0, 0)),
            pl.BlockSpec((1, hidden), lambda i, p: (0, 0)),
            pl.BlockSpec((1, hidden), lambda i, p: (0, 0)),
            pl.BlockSpec((1, periods), lambda i, p: (0, 0)),
        ],
        out_specs=pl.BlockSpec((BB, hidden), lambda i, p: (i, 0)),
        out_shape=jax.ShapeDtypeStruct((batch, hidden), jnp.float32),
    )(agent_obs, Wz, LzW, bz.reshape(1, hidden), Lzb.reshape(1, hidden),
      Wh, LhW, bh.reshape(1, hidden), Lhb.reshape(1, hidden),
      attention.reshape(1, periods))

    return jnp.concatenate(
        [x, hideout_obs.astype(jnp.float32), timestep_obs.astype(jnp.float32)],
        axis=-1)


# speedup vs baseline: 39.3982x; 39.3982x over previous
import jax
import jax.numpy as jnp
from jax.experimental import pallas as pl

BB = 8


def _body(x_ref, wz_ref, lzw_ref, bz_ref, lzb_ref, wh_ref, lhw_ref, bh_ref,
          lhb_ref, att_ref, o_ref):
    p = pl.program_id(1)
    n = x_ref.shape[2]
    hidden = lzw_ref.shape[1]

    lzw1 = lzw_ref[0:hidden, :]
    lhw1 = lhw_ref[0:hidden, :]
    gmz = jnp.dot(wz_ref[...], lzw1, preferred_element_type=jnp.float32)
    gmh = jnp.dot(wh_ref[...], lhw1, preferred_element_type=jnp.float32)
    gmat = jnp.concatenate([gmz, gmh], axis=1)
    gz = jnp.dot(bz_ref[...], lzw1) + lzb_ref[...]
    gh = jnp.dot(bh_ref[...], lhw1) + lhb_ref[...]

    r = jax.lax.broadcasted_iota(jnp.int32, (n, n), 0)
    c = jax.lax.broadcasted_iota(jnp.int32, (n, n), 1)
    rf = r.astype(jnp.float32)
    cf = c.astype(jnp.float32)
    adj = jnp.where(r >= c,
                    jax.lax.rsqrt(rf + 1.0) * jax.lax.rsqrt(cf + 1.0), 0.0)

    a = att_ref[...]
    e = jnp.exp(a - jnp.max(a))
    onehot = jax.lax.broadcasted_iota(jnp.int32, a.shape, 1) == p
    prob_p = jnp.sum(jnp.where(onehot, e, 0.0)) / jnp.sum(e)

    @pl.when(p == 0)
    def _init():
        o_ref[...] = jnp.zeros_like(o_ref)

    rows = []
    for b in range(BB):
        xb = x_ref[b, 0]
        ub = jnp.dot(xb, gmat, preferred_element_type=jnp.float32)
        pb = jnp.dot(adj, ub, preferred_element_type=jnp.float32)
        pz = pb[:, 0:hidden] + gz
        ph = pb[:, hidden:2 * hidden] + gh
        hp = (1.0 - jax.nn.sigmoid(pz)) * jnp.tanh(ph)
        rows.append(jnp.mean(hp, axis=0, keepdims=True))
    o_ref[...] += prob_p * jnp.concatenate(rows, axis=0)


def kernel(agent_obs, hideout_obs, timestep_obs, num_agents, edge_index,
           Wz, bz, Wr, br, Wh, bh, LzW, Lzb, LrW, Lrb, LhW, Lhb, attention):
    agent_obs = agent_obs.astype(jnp.float32)
    batch, periods, n, f = agent_obs.shape
    hidden = LzW.shape[1]

    x = pl.pallas_call(
        _body,
        grid=(batch // BB, periods),
        in_specs=[
            pl.BlockSpec((BB, 1, n, f), lambda i, p: (i, p, 0, 0)),
            pl.BlockSpec((f, hidden), lambda i, p: (0, 0)),
            pl.BlockSpec((2 * hidden, hidden), lambda i, p: (0, 0)),
            pl.BlockSpec((1, hidden), lambda i, p: (0, 0)),
            pl.BlockSpec((1, hidden), lambda i, p: (0, 0)),
            pl.BlockSpec((f, hidden), lambda i, p: (0, 0)),
            pl.BlockSpec((2 * hidden, hidden), lambda i, p: (0, 0)),
            pl.BlockSpec((1, hidden), lambda i, p: (0, 0)),
            pl.BlockSpec((1, hidden), lambda i, p: (0, 0)),
            pl.BlockSpec((1, periods), lambda i, p: (0, 0)),
        ],
        out_specs=pl.BlockSpec((BB, hidden), lambda i, p: (i, 0)),
        out_shape=jax.ShapeDtypeStruct((batch, hidden), jnp.float32),
    )(agent_obs, Wz, LzW, bz.reshape(1, hidden), Lzb.reshape(1, hidden),
      Wh, LhW, bh.reshape(1, hidden), Lhb.reshape(1, hidden),
      attention.reshape(1, periods))

    return jnp.concatenate(
        [x, hideout_obs.astype(jnp.float32), timestep_obs.astype(jnp.float32)],
        axis=-1)


# speedup vs baseline: 61.7230x; 1.5666x over previous
import jax
import jax.numpy as jnp
from jax.experimental import pallas as pl
from jax.experimental.pallas import tpu as pltpu

BB = 8


def _body(x_ref, wz_ref, lzw_ref, bz_ref, lzb_ref, wh_ref, lhw_ref, bh_ref,
          lhb_ref, att_ref, o_ref, adj_ref, gbd_ref, bias_ref, probs_ref):
    i = pl.program_id(0)
    p = pl.program_id(1)
    n = x_ref.shape[1]
    f = wz_ref.shape[0]
    hidden = lzw_ref.shape[1]
    zw = BB * hidden

    @pl.when((i == 0) & (p == 0))
    def _init_scratch():
        r = jax.lax.broadcasted_iota(jnp.int32, (n, n), 0)
        c = jax.lax.broadcasted_iota(jnp.int32, (n, n), 1)
        adj_ref[...] = jnp.where(
            r >= c,
            jax.lax.rsqrt(r.astype(jnp.float32) + 1.0) *
            jax.lax.rsqrt(c.astype(jnp.float32) + 1.0), 0.0)
        lzw1 = lzw_ref[0:hidden, :]
        lhw1 = lhw_ref[0:hidden, :]
        gmz = jnp.dot(wz_ref[...], lzw1, preferred_element_type=jnp.float32)
        gmh = jnp.dot(wh_ref[...], lhw1, preferred_element_type=jnp.float32)
        gbd_ref[...] = jnp.zeros_like(gbd_ref)
        for b in range(BB):
            gbd_ref[b * f:(b + 1) * f, b * hidden:(b + 1) * hidden] = gmz
            gbd_ref[b * f:(b + 1) * f, zw + b * hidden:zw + (b + 1) * hidden] = gmh
        gz = jnp.dot(bz_ref[...], lzw1) + lzb_ref[...]
        gh = jnp.dot(bh_ref[...], lhw1) + lhb_ref[...]
        bias_ref[...] = jnp.concatenate([gz] * BB + [gh] * BB, axis=1)
        a = att_ref[...]
        e = jnp.exp(a - jnp.max(a))
        probs_ref[...] = e / jnp.sum(e)

    onehot = jax.lax.broadcasted_iota(jnp.int32, probs_ref.shape, 1) == p
    prob_p = jnp.sum(jnp.where(onehot, probs_ref[...], 0.0))

    y = jnp.dot(adj_ref[...], x_ref[0],
                preferred_element_type=jnp.float32)
    pre = jnp.dot(y, gbd_ref[...],
                  preferred_element_type=jnp.float32) + bias_ref[...]
    hp = (1.0 - jax.nn.sigmoid(pre[:, 0:zw])) * jnp.tanh(pre[:, zw:2 * zw])
    xmean = jnp.mean(hp, axis=0, keepdims=True)

    @pl.when(p == 0)
    def _init_out():
        o_ref[...] = jnp.zeros_like(o_ref)

    o_ref[0] += prob_p * xmean


def kernel(agent_obs, hideout_obs, timestep_obs, num_agents, edge_index,
           Wz, bz, Wr, br, Wh, bh, LzW, Lzb, LrW, Lrb, LhW, Lhb, attention):
    agent_obs = agent_obs.astype(jnp.float32)
    batch, periods, n, f = agent_obs.shape
    hidden = LzW.shape[1]

    xt = jnp.transpose(agent_obs, (1, 2, 0, 3)).reshape(periods, n, batch * f)

    x = pl.pallas_call(
        _body,
        grid=(batch // BB, periods),
        in_specs=[
            pl.BlockSpec((1, n, BB * f), lambda i, p: (p, 0, i)),
            pl.BlockSpec((f, hidden), lambda i, p: (0, 0)),
            pl.BlockSpec((2 * hidden, hidden), lambda i, p: (0, 0)),
            pl.BlockSpec((1, hidden), lambda i, p: (0, 0)),
            pl.BlockSpec((1, hidden), lambda i, p: (0, 0)),
            pl.BlockSpec((f, hidden), lambda i, p: (0, 0)),
            pl.BlockSpec((2 * hidden, hidden), lambda i, p: (0, 0)),
            pl.BlockSpec((1, hidden), lambda i, p: (0, 0)),
            pl.BlockSpec((1, hidden), lambda i, p: (0, 0)),
            pl.BlockSpec((1, periods), lambda i, p: (0, 0)),
        ],
        out_specs=pl.BlockSpec((1, 1, BB * hidden), lambda i, p: (i, 0, 0)),
        out_shape=jax.ShapeDtypeStruct((batch // BB, 1, BB * hidden),
                                       jnp.float32),
        scratch_shapes=[
            pltpu.VMEM((n, n), jnp.float32),
            pltpu.VMEM((BB * f, 2 * BB * hidden), jnp.float32),
            pltpu.VMEM((1, 2 * BB * hidden), jnp.float32),
            pltpu.VMEM((1, periods), jnp.float32),
        ],
    )(xt, Wz, LzW, bz.reshape(1, hidden), Lzb.reshape(1, hidden),
      Wh, LhW, bh.reshape(1, hidden), Lhb.reshape(1, hidden),
      attention.reshape(1, periods))

    x = x.reshape(batch, hidden)
    return jnp.concatenate(
        [x, hideout_obs.astype(jnp.float32), timestep_obs.astype(jnp.float32)],
        axis=-1)


# speedup vs baseline: 156.7150x; 2.5390x over previous
import jax
import jax.numpy as jnp
from jax.experimental import pallas as pl
from jax.experimental.pallas import tpu as pltpu

BB = 8


def _body(x_ref, wz_ref, lzw_ref, bz_ref, lzb_ref, wh_ref, lhw_ref, bh_ref,
          lhb_ref, att_ref, o_ref, adj_ref, gbd_ref, bias_ref, probs_ref):
    i = pl.program_id(0)
    periods = x_ref.shape[0]
    n = x_ref.shape[1]
    f = wz_ref.shape[0]
    hidden = lzw_ref.shape[1]
    zw = BB * hidden

    @pl.when(i == 0)
    def _init_scratch():
        r = jax.lax.broadcasted_iota(jnp.int32, (n, n), 0)
        c = jax.lax.broadcasted_iota(jnp.int32, (n, n), 1)
        adj_ref[...] = jnp.where(
            r >= c,
            jax.lax.rsqrt(r.astype(jnp.float32) + 1.0) *
            jax.lax.rsqrt(c.astype(jnp.float32) + 1.0), 0.0)
        lzw1 = lzw_ref[0:hidden, :]
        lhw1 = lhw_ref[0:hidden, :]
        gmz = jnp.dot(wz_ref[...], lzw1, preferred_element_type=jnp.float32)
        gmh = jnp.dot(wh_ref[...], lhw1, preferred_element_type=jnp.float32)
        gbd_ref[...] = jnp.zeros_like(gbd_ref)
        for b in range(BB):
            gbd_ref[b * f:(b + 1) * f, b * hidden:(b + 1) * hidden] = gmz
            gbd_ref[b * f:(b + 1) * f, zw + b * hidden:zw + (b + 1) * hidden] = gmh
        gz = jnp.dot(bz_ref[...], lzw1) + lzb_ref[...]
        gh = jnp.dot(bh_ref[...], lhw1) + lhb_ref[...]
        bias_ref[...] = jnp.concatenate([gz] * BB + [gh] * BB, axis=1)
        a = att_ref[...]
        e = jnp.exp(a - jnp.max(a))
        probs_ref[...] = e / jnp.sum(e)

    adj = adj_ref[...]
    gbd = gbd_ref[...]
    bias = bias_ref[...]
    acc = jnp.zeros((1, zw), jnp.float32)
    for p in range(periods):
        y = jnp.dot(adj, x_ref[p], preferred_element_type=jnp.float32)
        pre = jnp.dot(y, gbd, preferred_element_type=jnp.float32) + bias
        hp = (1.0 - jax.nn.sigmoid(pre[:, 0:zw])) * jnp.tanh(pre[:, zw:2 * zw])
        xmean = jnp.sum(hp, axis=0, keepdims=True) * (1.0 / n)
        acc = acc + probs_ref[0, p] * xmean
    o_ref[0] = acc


def kernel(agent_obs, hideout_obs, timestep_obs, num_agents, edge_index,
           Wz, bz, Wr, br, Wh, bh, LzW, Lzb, LrW, Lrb, LhW, Lhb, attention):
    agent_obs = agent_obs.astype(jnp.float32)
    batch, periods, n, f = agent_obs.shape
    hidden = LzW.shape[1]

    xt = jnp.transpose(agent_obs, (1, 2, 0, 3)).reshape(periods, n, batch * f)

    x = pl.pallas_call(
        _body,
        grid=(batch // BB,),
        in_specs=[
            pl.BlockSpec((periods, n, BB * f), lambda i: (0, 0, i)),
            pl.BlockSpec((f, hidden), lambda i: (0, 0)),
            pl.BlockSpec((2 * hidden, hidden), lambda i: (0, 0)),
            pl.BlockSpec((1, hidden), lambda i: (0, 0)),
            pl.BlockSpec((1, hidden), lambda i: (0, 0)),
            pl.BlockSpec((f, hidden), lambda i: (0, 0)),
            pl.BlockSpec((2 * hidden, hidden), lambda i: (0, 0)),
            pl.BlockSpec((1, hidden), lambda i: (0, 0)),
            pl.BlockSpec((1, hidden), lambda i: (0, 0)),
            pl.BlockSpec((1, periods), lambda i: (0, 0)),
        ],
        out_specs=pl.BlockSpec((1, 1, BB * hidden), lambda i: (i, 0, 0)),
        out_shape=jax.ShapeDtypeStruct((batch // BB, 1, BB * hidden),
                                       jnp.float32),
        scratch_shapes=[
            pltpu.VMEM((n, n), jnp.float32),
            pltpu.VMEM((BB * f, 2 * BB * hidden), jnp.float32),
            pltpu.VMEM((1, 2 * BB * hidden), jnp.float32),
            pltpu.VMEM((1, periods), jnp.float32),
        ],
    )(xt, Wz, LzW, bz.reshape(1, hidden), Lzb.reshape(1, hidden),
      Wh, LhW, bh.reshape(1, hidden), Lhb.reshape(1, hidden),
      attention.reshape(1, periods))

    x = x.reshape(batch, hidden)
    return jnp.concatenate(
        [x, hideout_obs.astype(jnp.float32), timestep_obs.astype(jnp.float32)],
        axis=-1)
